# baseline (speedup 1.0000x reference)
import jax
import jax.numpy as jnp
from jax.experimental import pallas as pl

_B, _C, _H, _W = 8, 128, 128, 128
_F = 32
_P = 2
_HB = 64


def _gemm_kernel(idx_ref, w_ref, x_ref, o_ref):
    idxv = idx_ref[0, :]
    cols = jax.lax.broadcasted_iota(jnp.int32, (_C, _C), 1)
    onehot_t = (idxv[:, None] == cols).astype(jnp.float32)
    w_eff = jnp.dot(w_ref[...], onehot_t, preferred_element_type=jnp.float32)
    w_hi = w_eff.astype(jnp.bfloat16)
    zero = jnp.zeros((_F, _C), jnp.bfloat16)
    w2 = jnp.concatenate(
        [jnp.concatenate([w_hi if i == j else zero for j in range(_P)], axis=1)
         for i in range(_P)], axis=0)

    xt = jnp.swapaxes(x_ref[0].astype(jnp.bfloat16), 0, 1)
    xr = xt.reshape(_HB * _C, _W)
    outs = []
    for h2 in range(_HB // _P):
        seg = xr[h2 * _P * _C:(h2 + 1) * _P * _C, :]
        outs.append(jnp.dot(w2, seg, preferred_element_type=jnp.float32))
    ot = jnp.concatenate(outs, axis=0).reshape(_HB, _F, _W)
    o_ref[0] = jnp.swapaxes(ot, 0, 1)


def kernel(x, weights, channel_idx):
    w2 = weights.reshape(_F, _C)
    idx2 = channel_idx.reshape(1, _C)
    out = pl.pallas_call(
        _gemm_kernel,
        grid=(_B, _H // _HB),
        in_specs=[
            pl.BlockSpec((1, _C), lambda b, h: (0, 0)),
            pl.BlockSpec((_F, _C), lambda b, h: (0, 0)),
            pl.BlockSpec((1, _C, _HB, _W), lambda b, h: (b, 0, h, 0)),
        ],
        out_specs=pl.BlockSpec((1, _F, _HB, _W), lambda b, h: (b, 0, h, 0)),
        out_shape=jax.ShapeDtypeStruct((_B, _F, _H, _W), jnp.float32),
    )(idx2, w2, x)
    return out
